# Initial kernel scaffold; baseline (speedup 1.0000x reference)
import jax
import jax.numpy as jnp
from jax.experimental import pallas as pl


def kernel(x, table):
    raise NotImplementedError("write your pallas kernel here")


# speedup vs baseline: 1.1352x; 1.1352x over previous
import functools

import jax
import jax.numpy as jnp
from jax import lax
from jax.experimental import pallas as pl
from jax.experimental.pallas import tpu as pltpu
from jax.experimental.pallas import tpu_sc as plsc

MAX_NORM = 1.0
EPS = 1e-7

_NUM_CORES = 2
_NUM_SUBCORES = 16
_NUM_WORKERS = _NUM_CORES * _NUM_SUBCORES

_GATHER_WINDOW = 1024


def _sc_gather(table, idx_flat):
    n = idx_flat.shape[0]
    d = table.shape[1]
    idx2 = idx_flat.reshape(1, n)
    mesh = plsc.VectorSubcoreMesh(core_axis_name="c", subcore_axis_name="s")

    @functools.partial(
        pl.kernel,
        out_type=jax.ShapeDtypeStruct((n, d), table.dtype),
        mesh=mesh,
        compiler_params=pltpu.CompilerParams(use_tc_tiling_on_sc=False),
    )
    def gather_kernel(table_hbm, idx_hbm, out_hbm):
        def body(idx_vmem, out_vmem):
            pltpu.sync_copy(table_hbm.at[idx_vmem.at[0]], out_vmem)

        pltpu.emit_pipeline(
            body,
            grid=(n // _GATHER_WINDOW,),
            in_specs=[
                pl.BlockSpec((1, _GATHER_WINDOW), lambda i: (0, i)),
            ],
            out_specs=[
                pl.BlockSpec((_GATHER_WINDOW, d), lambda i: (i, 0)),
            ],
            core_axis_name=("c", "s"),
            dimension_semantics=(pltpu.PARALLEL,),
        )(idx_hbm, out_hbm)

    return gather_kernel(table, idx2)


_RENORM_ROWS = 4096


def _renorm_body(e_ref, o_ref):
    e = e_ref[...]
    sq = jnp.sum(e * e, axis=1, keepdims=True)
    norm = jnp.sqrt(sq)
    scale = jnp.where(norm > MAX_NORM, MAX_NORM / (norm + EPS), 1.0)
    o_ref[...] = e * scale


def _tc_renorm(emb):
    n, d = emb.shape
    return pl.pallas_call(
        _renorm_body,
        grid=(n // _RENORM_ROWS,),
        in_specs=[pl.BlockSpec((_RENORM_ROWS, d), lambda i: (i, 0))],
        out_specs=pl.BlockSpec((_RENORM_ROWS, d), lambda i: (i, 0)),
        out_shape=jax.ShapeDtypeStruct((n, d), emb.dtype),
    )(emb)


def kernel(x, table):
    b, l = x.shape
    d = table.shape[1]
    idx_flat = x.reshape(-1).astype(jnp.int32)
    emb = _sc_gather(table, idx_flat)
    out = _tc_renorm(emb)
    return out.reshape(b, l, d)
